# baseline (speedup 1.0000x reference)
import jax
import jax.numpy as jnp
from jax.experimental import pallas as pl

_N, _NC, _K = 4096, 512, 64
_BLK = 512
_CCH = 128
_NCH = _NC // _CCH
_IMASK = _NC - 1
_INTMAX = 2**31 - 1


def _assign_kernel(x_ref, mu_ref, out_ref):
    x = x_ref[...]
    xx = jnp.sum(x * x, axis=1, keepdims=True)
    packed = []
    for c in range(_NCH):
        m = mu_ref[pl.ds(c * _CCH, _CCH), :]
        g = jax.lax.dot_general(
            x, m, (((1,), (1,)), ((), ())),
            preferred_element_type=jnp.float32,
            precision=jax.lax.Precision.HIGHEST)
        mn = jnp.sum(m * m, axis=1)
        d = jnp.maximum((xx + mn[None, :]) - 2.0 * g, 0.0)
        bits = jax.lax.bitcast_convert_type(d, jnp.int32)
        iota = jax.lax.broadcasted_iota(jnp.int32, (_BLK, _CCH), 1) + c * _CCH
        packed.append((bits & ~_IMASK) | iota)
    p = packed[0]
    for q in packed[1:]:
        p = jnp.minimum(p, q)
    b1 = jnp.min(p, axis=1, keepdims=True)
    eq1 = p == b1
    p2 = jnp.where(eq1, _INTMAX, p)
    b2 = jnp.min(p2, axis=1, keepdims=True)
    eq2 = p2 == b2
    i1 = b1 & _IMASK
    i2 = b2 & _IMASK
    c1 = i1 & ~(_CCH - 1)
    c2 = i2 & ~(_CCH - 1)
    mu_a = jnp.zeros((_BLK, _K), jnp.float32)
    mu_b = jnp.zeros((_BLK, _K), jnp.float32)
    for c in range(_NCH):
        m = mu_ref[pl.ds(c * _CCH, _CCH), :]
        oh_a = (eq1 & (c1 == c * _CCH)).astype(jnp.float32)
        oh_b = (eq2 & (c2 == c * _CCH)).astype(jnp.float32)
        mu_a = mu_a + jax.lax.dot_general(
            oh_a, m, (((1,), (0,)), ((), ())),
            preferred_element_type=jnp.float32,
            precision=jax.lax.Precision.HIGHEST)
        mu_b = mu_b + jax.lax.dot_general(
            oh_b, m, (((1,), (0,)), ((), ())),
            preferred_element_type=jnp.float32,
            precision=jax.lax.Precision.HIGHEST)
    da = x - mu_a
    db = x - mu_b
    sa = jnp.sqrt(jnp.sum(da * da, axis=1, keepdims=True))
    sb = jnp.sqrt(jnp.sum(db * db, axis=1, keepdims=True))
    pick_a = (sa < sb) | ((sa == sb) & (i1 < i2))
    out_ref[...] = jnp.where(pick_a, i1, i2)[None]


def kernel(X, mu):
    mu2 = mu.reshape(_NC, _K)
    grid = _N // _BLK
    out = pl.pallas_call(
        _assign_kernel,
        grid=(grid,),
        in_specs=[
            pl.BlockSpec((_BLK, _K), lambda i: (i, 0)),
            pl.BlockSpec((_NC, _K), lambda i: (0, 0)),
        ],
        out_specs=pl.BlockSpec((1, _BLK, 1), lambda i: (i, 0, 0)),
        out_shape=jax.ShapeDtypeStruct((grid, _BLK, 1), jnp.int32),
    )(X, mu2)
    return out.reshape(_N)


# speedup vs baseline: 1.1206x; 1.1206x over previous
import jax
import jax.numpy as jnp
from jax.experimental import pallas as pl

_N, _NC, _K = 4096, 512, 64
_BLK = 1024
_CCH = 128
_NCH = _NC // _CCH
_IMASK = _NC - 1
_INTMAX = 2**31 - 1


def _assign_kernel(x_ref, mu_ref, out_ref):
    x = x_ref[...]
    xx = jnp.sum(x * x, axis=1, keepdims=True)
    packed = []
    for c in range(_NCH):
        m = mu_ref[pl.ds(c * _CCH, _CCH), :]
        g = jax.lax.dot_general(
            x, m, (((1,), (1,)), ((), ())),
            preferred_element_type=jnp.float32,
            precision=jax.lax.Precision.HIGHEST)
        mn = jnp.sum(m * m, axis=1)
        d = jnp.maximum((xx + mn[None, :]) - 2.0 * g, 0.0)
        bits = jax.lax.bitcast_convert_type(d, jnp.int32)
        iota = jax.lax.broadcasted_iota(jnp.int32, (_BLK, _CCH), 1) + c * _CCH
        packed.append((bits & ~_IMASK) | iota)
    p = packed[0]
    for q in packed[1:]:
        p = jnp.minimum(p, q)
    b1 = jnp.min(p, axis=1, keepdims=True)
    p2 = jnp.where(p == b1, _INTMAX, p)
    b2 = jnp.min(p2, axis=1, keepdims=True)
    i1 = b1 & _IMASK
    i2 = b2 & _IMASK
    mu_a = jnp.zeros((_BLK, _K), jnp.float32)
    mu_b = jnp.zeros((_BLK, _K), jnp.float32)
    for c in range(_NCH):
        m = mu_ref[pl.ds(c * _CCH, _CCH), :]
        iota = jax.lax.broadcasted_iota(jnp.int32, (_BLK, _CCH), 1) + c * _CCH
        oh_a = (iota == i1).astype(jnp.float32)
        oh_b = (iota == i2).astype(jnp.float32)
        mu_a = mu_a + jax.lax.dot_general(
            oh_a, m, (((1,), (0,)), ((), ())),
            preferred_element_type=jnp.float32,
            precision=jax.lax.Precision.HIGHEST)
        mu_b = mu_b + jax.lax.dot_general(
            oh_b, m, (((1,), (0,)), ((), ())),
            preferred_element_type=jnp.float32,
            precision=jax.lax.Precision.HIGHEST)
    da = x - mu_a
    db = x - mu_b
    sa = jnp.sqrt(jnp.sum(da * da, axis=1, keepdims=True))
    sb = jnp.sqrt(jnp.sum(db * db, axis=1, keepdims=True))
    pick_a = (sa < sb) | ((sa == sb) & (i1 < i2))
    out_ref[...] = jnp.where(pick_a, i1, i2)[None]


def kernel(X, mu):
    mu2 = mu.reshape(_NC, _K)
    grid = _N // _BLK
    out = pl.pallas_call(
        _assign_kernel,
        grid=(grid,),
        in_specs=[
            pl.BlockSpec((_BLK, _K), lambda i: (i, 0)),
            pl.BlockSpec((_NC, _K), lambda i: (0, 0)),
        ],
        out_specs=pl.BlockSpec((1, _BLK, 1), lambda i: (i, 0, 0)),
        out_shape=jax.ShapeDtypeStruct((grid, _BLK, 1), jnp.int32),
    )(X, mu2)
    return out.reshape(_N)
